# baseline (speedup 1.0000x reference)
import jax
import jax.numpy as jnp
from jax import lax
from jax.experimental import pallas as pl
from jax.experimental.pallas import tpu as pltpu
from jax.experimental.pallas import tpu_sc as plsc

D = 128
B = 320000
NC, NS = 2, 16
NW = NC * NS
BPW = B // NW
C = 40
NCHUNK = BPW // C
NBUF = 10
AHEAD = 4


def _mixer_body(x_hbm, ei_hbm, out_hbm, idx_s, idx_d, *bufs):
    A = bufs[0:NBUF]
    Bv = bufs[NBUF:2 * NBUF]
    GS = bufs[2 * NBUF:3 * NBUF]
    WS = bufs[3 * NBUF:4 * NBUF]

    wid = lax.axis_index("s") * NC + lax.axis_index("c")
    base_w = wid * BPW
    pltpu.sync_copy(ei_hbm.at[pl.ds(base_w, BPW)], idx_s)
    pltpu.sync_copy(ei_hbm.at[pl.ds(B + base_w, BPW)], idx_d)

    def issue_gather(h, k):
        off = h * C
        pltpu.async_copy(x_hbm.at[idx_s.at[pl.ds(off, C)]], A[k], GS[k])
        pltpu.async_copy(x_hbm.at[idx_d.at[pl.ds(off, C)]], Bv[k], GS[k])

    def drain_gather(h, k):
        off = h * C
        pltpu.make_async_copy(x_hbm.at[idx_s.at[pl.ds(off, C)]], A[k], GS[k]).wait()
        pltpu.make_async_copy(x_hbm.at[idx_d.at[pl.ds(off, C)]], Bv[k], GS[k]).wait()

    def issue_write(h, k):
        pltpu.async_copy(A[k], out_hbm.at[pl.ds(base_w + h * C, C)], WS[k])

    def drain_write(h, k):
        pltpu.make_async_copy(A[k], out_hbm.at[pl.ds(base_w + h * C, C)], WS[k]).wait()

    for h in range(AHEAD):
        issue_gather(h, h)

    def outer(o, carry):
        for k in range(NBUF):
            g = o * NBUF + k
            j = (k + AHEAD) % NBUF

            @pl.when(g >= NBUF - AHEAD)
            def _():
                drain_write(g - (NBUF - AHEAD), j)

            @pl.when(g + AHEAD < NCHUNK)
            def _():
                issue_gather(g + AHEAD, j)

            drain_gather(g, k)

            @plsc.parallel_loop(0, C, unroll=8)
            def _(i):
                for t in range(D // 16):
                    sl = pl.ds(t * 16, 16)
                    A[k][i, sl] = A[k][i, sl] - Bv[k][i, sl]

            issue_write(g, k)
        return carry

    lax.fori_loop(0, NCHUNK // NBUF, outer, 0)

    for h in range(NCHUNK - (NBUF - AHEAD), NCHUNK):
        drain_write(h, h % NBUF)


def kernel(x, edge_index):
    mesh = plsc.VectorSubcoreMesh(core_axis_name="c", subcore_axis_name="s")
    run = pl.kernel(
        _mixer_body,
        out_type=jax.ShapeDtypeStruct((B, D), jnp.float32),
        mesh=mesh,
        scratch_types=[
            pltpu.VMEM((BPW,), jnp.int32),
            pltpu.VMEM((BPW,), jnp.int32),
        ] + [pltpu.VMEM((C, D), jnp.float32)] * (2 * NBUF)
          + [pltpu.SemaphoreType.DMA] * (2 * NBUF),
    )
    return run(x, edge_index.reshape(2 * B))


# speedup vs baseline: 1.0331x; 1.0331x over previous
import jax
import jax.numpy as jnp
from jax import lax
from jax.experimental import pallas as pl
from jax.experimental.pallas import tpu as pltpu
from jax.experimental.pallas import tpu_sc as plsc

D = 128
B = 320000
NC, NS = 2, 16
NW = NC * NS
BPW = B // NW
C = 80
NCHUNK = BPW // C
NBUF = 5
AHEAD = 2


def _mixer_body(x_hbm, ei_hbm, out_hbm, idx_s, idx_d,
                a0, a1, a2, a3, a4, b0, b1, b2, b3, b4,
                gs0, gs1, gs2, gs3, gs4, ws0, ws1, ws2, ws3, ws4):
    A = (a0, a1, a2, a3, a4)
    Bv = (b0, b1, b2, b3, b4)
    GS = (gs0, gs1, gs2, gs3, gs4)
    WS = (ws0, ws1, ws2, ws3, ws4)

    wid = lax.axis_index("s") * NC + lax.axis_index("c")
    base_w = wid * BPW
    pltpu.sync_copy(ei_hbm.at[pl.ds(base_w, BPW)], idx_s)
    pltpu.sync_copy(ei_hbm.at[pl.ds(B + base_w, BPW)], idx_d)

    def issue_gather(h, k):
        off = h * C
        pltpu.async_copy(x_hbm.at[idx_s.at[pl.ds(off, C)]], A[k], GS[k])
        pltpu.async_copy(x_hbm.at[idx_d.at[pl.ds(off, C)]], Bv[k], GS[k])

    def drain_gather(h, k):
        off = h * C
        pltpu.make_async_copy(x_hbm.at[idx_s.at[pl.ds(off, C)]], A[k], GS[k]).wait()
        pltpu.make_async_copy(x_hbm.at[idx_d.at[pl.ds(off, C)]], Bv[k], GS[k]).wait()

    def issue_write(h, k):
        pltpu.async_copy(A[k], out_hbm.at[pl.ds(base_w + h * C, C)], WS[k])

    def drain_write(h, k):
        pltpu.make_async_copy(A[k], out_hbm.at[pl.ds(base_w + h * C, C)], WS[k]).wait()

    for h in range(AHEAD):
        issue_gather(h, h)

    def outer(o, carry):
        for k in range(NBUF):
            g = o * NBUF + k
            j = (k + AHEAD) % NBUF

            @pl.when(g >= NBUF - AHEAD)
            def _():
                drain_write(g - (NBUF - AHEAD), j)

            @pl.when(g + AHEAD < NCHUNK)
            def _():
                issue_gather(g + AHEAD, j)

            drain_gather(g, k)

            @plsc.parallel_loop(0, C, unroll=8)
            def _(i):
                for t in range(D // 16):
                    sl = pl.ds(t * 16, 16)
                    A[k][i, sl] = A[k][i, sl] - Bv[k][i, sl]

            issue_write(g, k)
        return carry

    lax.fori_loop(0, NCHUNK // NBUF, outer, 0)

    for h in range(NCHUNK - (NBUF - AHEAD), NCHUNK):
        drain_write(h, h % NBUF)


def kernel(x, edge_index):
    mesh = plsc.VectorSubcoreMesh(core_axis_name="c", subcore_axis_name="s")
    run = pl.kernel(
        _mixer_body,
        out_type=jax.ShapeDtypeStruct((B, D), jnp.float32),
        mesh=mesh,
        scratch_types=[
            pltpu.VMEM((BPW,), jnp.int32),
            pltpu.VMEM((BPW,), jnp.int32),
        ] + [pltpu.VMEM((C, D), jnp.float32)] * (2 * NBUF)
          + [pltpu.SemaphoreType.DMA] * (2 * NBUF),
    )
    return run(x, edge_index.reshape(2 * B))
